# baseline (speedup 1.0000x reference)
import functools

import jax
import jax.numpy as jnp
from jax import lax
from jax.experimental import pallas as pl
from jax.experimental.pallas import tpu as pltpu
from jax.experimental.pallas import tpu_sc as plsc

B, C, H, W = 8, 96, 224, 224
HW = H * W
ROWS = B * C
HALF = ROWS // 2
TOT_H = HALF * HW
NW = 32
PER = TOT_H // NW
CHUNK = 25088
NCH = PER // CHUNK

R_BLK = 192
C_BLK = 12544
N_R = HALF // R_BLK
N_C = HW // (2 * C_BLK)


def _blk_argmax(x, j):
    m = jnp.max(x, axis=-1, keepdims=True)
    col = lax.broadcasted_iota(jnp.int32, x.shape, 1)
    big = jnp.int32(2**31 - 1)
    cand = jnp.min(jnp.where(x == m, col, big), axis=-1, keepdims=True)
    return m, cand + j * C_BLK


def _argmax_body(x1_ref, x2_ref, idx_ref, max_sc):
    j = pl.program_id(1)
    m1, cand1 = _blk_argmax(x1_ref[...], 2 * j)
    m2, cand2 = _blk_argmax(x2_ref[...], 2 * j + 1)
    two = m2 > m1
    m = jnp.where(two, m2, m1)
    cand = jnp.where(two, cand2, cand1)

    @pl.when(j == 0)
    def _():
        max_sc[...] = m
        idx_ref[0] = cand

    @pl.when(j != 0)
    def _():
        prev = max_sc[...]
        better = m > prev
        idx_ref[0] = jnp.where(better, cand, idx_ref[0])
        max_sc[...] = jnp.where(better, m, prev)


def _rowwise_argmax(flat):
    idx3 = pl.pallas_call(
        _argmax_body,
        grid=(N_R, N_C),
        in_specs=[
            pl.BlockSpec((R_BLK, C_BLK), lambda i, j: (i, 2 * j)),
            pl.BlockSpec((R_BLK, C_BLK), lambda i, j: (i, 2 * j + 1)),
        ],
        out_specs=pl.BlockSpec((1, R_BLK, 1), lambda i, j: (i, 0, 0)),
        out_shape=jax.ShapeDtypeStruct((N_R, R_BLK, 1), jnp.int32),
        scratch_shapes=[pltpu.VMEM((R_BLK, 1), jnp.float32)],
        compiler_params=pltpu.CompilerParams(
            dimension_semantics=("parallel", "arbitrary"),
        ),
    )(flat, flat)
    return idx3.reshape(HALF)


def _stream_body(x_hbm, out_hbm, buf, sem0, sem1):
    cid = lax.axis_index("c")
    sid = lax.axis_index("s")
    wid = sid * 2 + cid
    rows_per = HALF // NW
    row0 = HALF + wid * rows_per
    sems = (sem0, sem1)
    cps = []
    k = 0
    for r in range(rows_per):
        for h in range(2):
            cp = pltpu.async_copy(
                x_hbm.at[row0 + r, pl.ds(h * CHUNK, CHUNK)],
                buf.at[k % 2],
                sems[k % 2],
            )
            cps.append(cp)
            if k >= 1:
                cps[k - 1].wait()
            k += 1
    cps[-1].wait()

    @pl.when(wid == 0)
    def _():
        pltpu.sync_copy(buf.at[0, pl.ds(0, 16)], out_hbm)


@functools.cache
def _stream_sc():
    return pl.kernel(
        _stream_body,
        out_type=jax.ShapeDtypeStruct((16,), jnp.float32),
        mesh=plsc.VectorSubcoreMesh(core_axis_name="c", subcore_axis_name="s"),
        scratch_types=[
            pltpu.VMEM((2, CHUNK), jnp.float32),
            pltpu.SemaphoreType.DMA,
            pltpu.SemaphoreType.DMA,
        ],
        compiler_params=pltpu.CompilerParams(
            use_tc_tiling_on_sc=False, needs_layout_passes=False
        ),
    )


@jax.jit
def kernel(grid, heatmaps):
    flat = heatmaps.reshape(ROWS, HW)
    idx = _rowwise_argmax(flat[:HALF])
    probe = _stream_sc()(flat)
    out = jnp.zeros((B, C, 2), jnp.float32) + probe[0] + idx[0].astype(jnp.float32)
    return out


# speedup vs baseline: 1.7428x; 1.7428x over previous
import functools

import jax
import jax.numpy as jnp
from jax import lax
from jax.experimental import pallas as pl
from jax.experimental.pallas import tpu as pltpu
from jax.experimental.pallas import tpu_sc as plsc

B, C, H, W = 8, 96, 224, 224
HW = H * W
ROWS = B * C

R_BLK = 192
C_BLK = 12544
N_R = ROWS // R_BLK
N_C = HW // (2 * C_BLK)


def _blk_argmax(x, j):
    m = jnp.max(x, axis=-1, keepdims=True)
    col = lax.broadcasted_iota(jnp.int32, x.shape, 1)
    big = jnp.int32(2**31 - 1)
    cand = jnp.min(jnp.where(x == m, col, big), axis=-1, keepdims=True)
    return m, cand + j * C_BLK


def _argmax_body(x1_ref, x2_ref, idx_ref, max_sc):
    j = pl.program_id(1)
    m1, cand1 = _blk_argmax(x1_ref[...], 2 * j)
    m2, cand2 = _blk_argmax(x2_ref[...], 2 * j + 1)
    two = m2 > m1
    m = jnp.where(two, m2, m1)
    cand = jnp.where(two, cand2, cand1)

    @pl.when(j == 0)
    def _():
        max_sc[...] = m
        idx_ref[0] = cand

    @pl.when(j != 0)
    def _():
        prev = max_sc[...]
        better = m > prev
        idx_ref[0] = jnp.where(better, cand, idx_ref[0])
        max_sc[...] = jnp.where(better, m, prev)


def _rowwise_argmax(flat):
    idx3 = pl.pallas_call(
        _argmax_body,
        grid=(N_R, N_C),
        in_specs=[
            pl.BlockSpec((R_BLK, C_BLK), lambda i, j: (i, 2 * j)),
            pl.BlockSpec((R_BLK, C_BLK), lambda i, j: (i, 2 * j + 1)),
        ],
        out_specs=pl.BlockSpec((1, R_BLK, 1), lambda i, j: (i, 0, 0)),
        out_shape=jax.ShapeDtypeStruct((N_R, R_BLK, 1), jnp.int32),
        scratch_shapes=[pltpu.VMEM((R_BLK, 1), jnp.float32)],
        compiler_params=pltpu.CompilerParams(
            dimension_semantics=("parallel", "arbitrary"),
        ),
    )(flat, flat)
    return idx3.reshape(ROWS)


def _gather_body(tab_hbm, idx_hbm, out_hbm, idx_v, pos_v, rows_v, out_v, sem):
    cid = lax.axis_index("c")
    sid = lax.axis_index("s")
    wid = sid * 2 + cid
    lane = lax.broadcasted_iota(jnp.int32, (16,), 0)

    @pl.when(wid < B)
    def _():
        b = wid
        pltpu.sync_copy(idx_hbm.at[pl.ds(b * C, C)], idx_v)
        for plane in (0, 1):
            rowbase = (b * 2 + plane) * (HW // 8)
            for g in range(C // 16):
                iv = idx_v[pl.ds(g * 16, 16)]
                pos_v[pl.ds(g * 16, 16)] = rowbase + (iv >> 3)
            pltpu.async_copy(tab_hbm.at[pos_v], rows_v, sem).wait()
            oplane = 1 - plane
            for g in range(C // 16):
                iv = idx_v[pl.ds(g * 16, 16)]
                vals = plsc.load_gather(rows_v, [lane + g * 16, iv & 7])
                out_v[pl.ds(oplane * C + g * 16, 16)] = vals
        pltpu.sync_copy(out_v, out_hbm.at[b])


@functools.cache
def _gather_sc():
    return pl.kernel(
        _gather_body,
        out_type=jax.ShapeDtypeStruct((B, 2 * C), jnp.float32),
        mesh=plsc.VectorSubcoreMesh(core_axis_name="c", subcore_axis_name="s"),
        scratch_types=[
            pltpu.VMEM((C,), jnp.int32),
            pltpu.VMEM((C,), jnp.int32),
            pltpu.VMEM((C, 8), jnp.float32),
            pltpu.VMEM((2 * C,), jnp.float32),
            pltpu.SemaphoreType.DMA,
        ],
        compiler_params=pltpu.CompilerParams(
            use_tc_tiling_on_sc=False, needs_layout_passes=False
        ),
    )


@jax.jit
def kernel(grid, heatmaps):
    flat = heatmaps.reshape(ROWS, HW)
    idx = _rowwise_argmax(flat)
    table = grid.reshape(B * 2 * HW // 8, 8)
    out = _gather_sc()(table, idx)
    return out.reshape(B, 2, C).transpose(0, 2, 1)
